# baseline (speedup 1.0000x reference)
import functools

import jax
import jax.numpy as jnp
import numpy as np
from jax import lax
from jax.experimental import pallas as pl
from jax.experimental.pallas import tpu as pltpu
from jax.experimental.pallas import tpu_sc as plsc

N = 4096
B = 16
F = 512
K = 512
N_CHUNK = 4096
LANES = 16
NTILE = 16
EPT = N // NTILE
VPT = EPT // LANES
OUT_PAD = K + 2 * LANES

MIN_I32 = np.int32(-2147483648)
MSK_I32 = np.int32(2147483647)


def _reduce_body(w_ref, h_ref, o_ref):
    j = pl.program_id(1)
    part = jnp.sum(h_ref[0], axis=1, keepdims=True)

    @pl.when(j == 0)
    def _():
        o_ref[...] = part

    @pl.when(j > 0)
    def _():
        o_ref[...] = o_ref[...] + part

    @pl.when(j == B - 1)
    def _():
        w0 = w_ref[0]
        o_ref[...] = o_ref[...] * (w0 / jnp.sqrt(w0 * w0))


def _scores_tc(H, w):
    return pl.pallas_call(
        _reduce_body,
        grid=(N // N_CHUNK, B),
        in_specs=[
            pl.BlockSpec(memory_space=pltpu.SMEM),
            pl.BlockSpec((1, N_CHUNK, F), lambda i, j: (j, i, 0)),
        ],
        out_specs=pl.BlockSpec((N_CHUNK, 1), lambda i, j: (i, 0)),
        out_shape=jax.ShapeDtypeStruct((N, 1), jnp.float32),
    )(w.reshape(1), H)


def _topk_body(score_hbm, out_hbm,
               score_v, keys_v, lhist_v, hback_v, zeros_v, row_v, cidx_v,
               cnts_v, ident_lo, ident_hi, idxa_v, idxb_v, vala_v, valb_v,
               h_sp0, h_sp1, h_sp2, h_sp3, cnts_sp):
    cid = lax.axis_index("c")
    sid = lax.axis_index("s")
    h_sps = [h_sp0, h_sp1, h_sp2, h_sp3]

    @pl.when(cid == 0)
    def _work():
        wid = sid
        base = wid * EPT
        iota = lax.iota(jnp.int32, LANES)
        ones = jnp.ones((LANES,), jnp.int32)

        def _init(i, _):
            zeros_v[pl.ds(i * LANES, LANES)] = jnp.zeros((LANES,), jnp.int32)
            return 0

        lax.fori_loop(0, VPT, _init, 0)

        def _idn(i, _):
            ident_lo[pl.ds(i * LANES, LANES)] = iota + i * LANES
            ident_hi[pl.ds(i * LANES, LANES)] = iota + i * LANES + 128
            return 0

        lax.fori_loop(0, 8, _idn, 0)

        @pl.when(wid == 0)
        def _():
            for p in range(4):
                pltpu.sync_copy(zeros_v, h_sps[p])
            pltpu.sync_copy(zeros_v.at[pl.ds(0, 3 * NTILE)], cnts_sp)

        pltpu.sync_copy(score_hbm.at[pl.ds(base, EPT)], score_v)

        def _mk(i, _):
            f = score_v[pl.ds(i * LANES, LANES)]
            u = lax.bitcast_convert_type(f, jnp.int32)
            m = lax.shift_right_arithmetic(u, 31)
            keys_v[pl.ds(i * LANES, LANES)] = u ^ (m & MSK_I32) ^ MIN_I32
            return 0

        lax.fori_loop(0, VPT, _mk, 0)
        plsc.subcore_barrier()

        prefix = jnp.int32(0)
        k_rem = jnp.int32(K)
        for p in range(4):
            shift = 24 - 8 * p

            def _zl(i, _):
                lhist_v[pl.ds(i * LANES, LANES)] = jnp.zeros((LANES,),
                                                             jnp.int32)
                return 0

            lax.fori_loop(0, VPT, _zl, 0)

            def _scan(i, carry, shift=shift, p=p):
                kb = keys_v[pl.ds(i * LANES, LANES)]
                bucket = lax.shift_right_logical(kb, shift) & 255
                if p == 0:
                    plsc.addupdate_scatter(lhist_v, [bucket], ones)
                else:
                    hi = lax.shift_right_logical(kb, shift + 8)
                    plsc.addupdate_scatter(lhist_v, [bucket], ones,
                                           mask=hi == carry[0])
                return carry

            lax.fori_loop(0, VPT, _scan, (prefix,))

            pltpu.sync_copy(lhist_v.at[pl.ds(0, 128)],
                            h_sps[p].at[ident_lo], add=True)
            pltpu.sync_copy(lhist_v.at[pl.ds(128, 128)],
                            h_sps[p].at[ident_hi], add=True)
            plsc.subcore_barrier()
            pltpu.sync_copy(h_sps[p], hback_v)

            def _solve(i, carry):
                accum, found, tb, g_above = carry
                j = 15 - i
                hv = hback_v[pl.ds(j * LANES, LANES)]
                rev = lax.rev(hv, (0,))
                c = plsc.cumsum(rev)
                cond = (accum + c) >= k_rem
                npos = jnp.sum(cond.astype(jnp.int32))
                l = jnp.max(plsc.all_reduce_ffs(cond))
                c_l = jnp.sum(jnp.where(iota == l, c, 0))
                h_l = jnp.sum(jnp.where(iota == (15 - l), hv, 0))
                hit = jnp.logical_and(found == 0, npos > 0)
                tb = jnp.where(hit, j * LANES + 15 - l, tb)
                g_above = jnp.where(hit, accum + c_l - h_l, g_above)
                found = jnp.where(hit, jnp.int32(1), found)
                accum = accum + jnp.sum(hv)
                return accum, found, tb, g_above

            _, _, tb, g_above = lax.fori_loop(
                0, 16, _solve,
                (jnp.int32(0), jnp.int32(0), jnp.int32(0), jnp.int32(0)))
            if p == 0:
                prefix = tb
            else:
                prefix = lax.shift_left(prefix, 8) | tb
            k_rem = k_rem - g_above

        t_b = prefix
        t_s = t_b ^ MIN_I32

        def _cnt(i, carry):
            cg, ce = carry
            kb = keys_v[pl.ds(i * LANES, LANES)]
            gt = (kb ^ MIN_I32) > t_s
            eq = kb == t_b
            return (cg + jnp.sum(gt.astype(jnp.int32)),
                    ce + jnp.sum(eq.astype(jnp.int32)))

        cg, ce = lax.fori_loop(0, VPT, _cnt, (jnp.int32(0), jnp.int32(0)))
        row_v[...] = jnp.where(iota == 0, cg, jnp.where(iota == 1, ce, 0))
        cidx_v[...] = jnp.where(iota == 0, wid,
                                jnp.where(iota == 1, NTILE + wid,
                                          2 * NTILE + iota))
        pltpu.sync_copy(row_v, cnts_sp.at[cidx_v], add=True)
        plsc.subcore_barrier()
        pltpu.sync_copy(cnts_sp, cnts_v)

        cntg = cnts_v[pl.ds(0, LANES)]
        cnte = cnts_v[pl.ds(LANES, LANES)]
        need_eq = jnp.int32(K) - jnp.sum(cntg)
        gt_before = jnp.sum(jnp.where(iota == wid, plsc.cumsum(cntg), 0)) - cg
        eq_before = jnp.sum(jnp.where(iota == wid, plsc.cumsum(cnte), 0)) - ce
        my_off = gt_before + jnp.minimum(eq_before, need_eq)
        my_eq_budget = jnp.clip(need_eq - eq_before, 0, ce)
        trash = jnp.int32(K) + wid

        def _tgt(i, carry, buf_i, buf_v, voff):
            off, eq_taken = carry
            kb = keys_v[pl.ds((i + voff) * LANES, LANES)]
            gt = (kb ^ MIN_I32) > t_s
            eq = kb == t_b
            eqc = plsc.cumsum(eq.astype(jnp.int32))
            take_eq = jnp.logical_and(eq, eqc + eq_taken <= my_eq_budget)
            m = jnp.logical_or(gt, take_eq)
            mi = m.astype(jnp.int32)
            rank = plsc.cumsum(mi)
            buf_i[pl.ds(i * LANES, LANES)] = jnp.where(m, off + rank - 1,
                                                       trash)
            buf_v[pl.ds(i * LANES, LANES)] = iota + (base + (i + voff) * LANES)
            return (off + jnp.sum(mi),
                    eq_taken + jnp.sum(take_eq.astype(jnp.int32)))

        carry = lax.fori_loop(
            0, 8, functools.partial(_tgt, buf_i=idxa_v, buf_v=vala_v, voff=0),
            (my_off, jnp.int32(0)))
        lax.fori_loop(
            0, 8, functools.partial(_tgt, buf_i=idxb_v, buf_v=valb_v, voff=8),
            carry)

        pltpu.sync_copy(vala_v, out_hbm.at[idxa_v])
        pltpu.sync_copy(valb_v, out_hbm.at[idxb_v])


@functools.lru_cache(maxsize=None)
def _get_topk_sc():
    return pl.kernel(
        _topk_body,
        out_type=jax.ShapeDtypeStruct((OUT_PAD,), jnp.int32),
        mesh=plsc.VectorSubcoreMesh(core_axis_name="c", subcore_axis_name="s"),
        compiler_params=pltpu.CompilerParams(needs_layout_passes=False),
        scratch_types=[
            pltpu.VMEM((EPT,), jnp.float32),
            pltpu.VMEM((EPT,), jnp.int32),
            pltpu.VMEM((256,), jnp.int32),
            pltpu.VMEM((256,), jnp.int32),
            pltpu.VMEM((256,), jnp.int32),
            pltpu.VMEM((LANES,), jnp.int32),
            pltpu.VMEM((LANES,), jnp.int32),
            pltpu.VMEM((3 * NTILE,), jnp.int32),
            pltpu.VMEM((128,), jnp.int32),
            pltpu.VMEM((128,), jnp.int32),
            pltpu.VMEM((128,), jnp.int32),
            pltpu.VMEM((128,), jnp.int32),
            pltpu.VMEM((128,), jnp.int32),
            pltpu.VMEM((128,), jnp.int32),
            pltpu.VMEM_SHARED((256,), jnp.int32),
            pltpu.VMEM_SHARED((256,), jnp.int32),
            pltpu.VMEM_SHARED((256,), jnp.int32),
            pltpu.VMEM_SHARED((256,), jnp.int32),
            pltpu.VMEM_SHARED((3 * NTILE,), jnp.int32),
        ],
    )


def kernel(H, w):
    score = _scores_tc(H, jnp.asarray(w, jnp.float32)).reshape(N)
    return _get_topk_sc()(score)[:K]


# speedup vs baseline: 8.9981x; 8.9981x over previous
import functools

import jax
import jax.numpy as jnp
import numpy as np
from jax import lax
from jax.experimental import pallas as pl
from jax.experimental.pallas import tpu as pltpu
from jax.experimental.pallas import tpu_sc as plsc

N = 4096
B = 16
F = 512
K = 512
N_CHUNK = 4096
LANES = 16
NTILE = 16
EPT = N // NTILE
VPT = EPT // LANES
OUT_PAD = K + 2 * LANES

MIN_I32 = np.int32(-2147483648)
MSK_I32 = np.int32(2147483647)


def _reduce_body(w_ref, h_ref, o_ref):
    j = pl.program_id(1)
    part = jnp.sum(h_ref[0], axis=1, keepdims=True)

    @pl.when(j == 0)
    def _():
        o_ref[...] = part

    @pl.when(j > 0)
    def _():
        o_ref[...] = o_ref[...] + part

    @pl.when(j == B - 1)
    def _():
        w0 = w_ref[0]
        o_ref[...] = o_ref[...] * (w0 / jnp.sqrt(w0 * w0))


def _scores_tc(H, w):
    return pl.pallas_call(
        _reduce_body,
        grid=(N // N_CHUNK, B),
        in_specs=[
            pl.BlockSpec(memory_space=pltpu.SMEM),
            pl.BlockSpec((1, N_CHUNK, F), lambda i, j: (j, i, 0)),
        ],
        out_specs=pl.BlockSpec((N_CHUNK, 1), lambda i, j: (i, 0)),
        out_shape=jax.ShapeDtypeStruct((N, 1), jnp.float32),
    )(w.reshape(1), H)


def _topk_body(score_hbm, out_hbm,
               score_v, keys_v, lhist_v, hback_v, zeros_v, row_v, cidx_v,
               cnts_v, ident_lo, ident_hi, idxa_v, idxb_v, vala_v, valb_v,
               h_sp0, h_sp1, h_sp2, h_sp3, cnts_sp, out_sp):
    cid = lax.axis_index("c")
    sid = lax.axis_index("s")
    h_sps = [h_sp0, h_sp1, h_sp2, h_sp3]

    @pl.when(cid == 0)
    def _work():
        wid = sid
        base = wid * EPT
        iota = lax.iota(jnp.int32, LANES)
        ones = jnp.ones((LANES,), jnp.int32)

        def _init(i, _):
            zeros_v[pl.ds(i * LANES, LANES)] = jnp.zeros((LANES,), jnp.int32)
            return 0

        lax.fori_loop(0, VPT, _init, 0)

        def _idn(i, _):
            ident_lo[pl.ds(i * LANES, LANES)] = iota + i * LANES
            ident_hi[pl.ds(i * LANES, LANES)] = iota + i * LANES + 128
            return 0

        lax.fori_loop(0, 8, _idn, 0)

        @pl.when(wid == 0)
        def _():
            for p in range(4):
                pltpu.sync_copy(zeros_v, h_sps[p])
            pltpu.sync_copy(zeros_v.at[pl.ds(0, 3 * NTILE)], cnts_sp)

        pltpu.sync_copy(score_hbm.at[pl.ds(base, EPT)], score_v)

        def _mk(i, _):
            f = score_v[pl.ds(i * LANES, LANES)]
            u = lax.bitcast_convert_type(f, jnp.int32)
            m = lax.shift_right_arithmetic(u, 31)
            keys_v[pl.ds(i * LANES, LANES)] = u ^ (m & MSK_I32) ^ MIN_I32
            return 0

        lax.fori_loop(0, VPT, _mk, 0)
        plsc.subcore_barrier()

        prefix = jnp.int32(0)
        k_rem = jnp.int32(K)
        for p in range(4):
            shift = 24 - 8 * p

            def _zl(i, _):
                lhist_v[pl.ds(i * LANES, LANES)] = jnp.zeros((LANES,),
                                                             jnp.int32)
                return 0

            lax.fori_loop(0, VPT, _zl, 0)

            def _scan(i, carry, shift=shift, p=p):
                kb = keys_v[pl.ds(i * LANES, LANES)]
                bucket = lax.shift_right_logical(kb, shift) & 255
                if p == 0:
                    plsc.addupdate_scatter(lhist_v, [bucket], ones)
                else:
                    hi = lax.shift_right_logical(kb, shift + 8)
                    plsc.addupdate_scatter(lhist_v, [bucket], ones,
                                           mask=hi == carry[0])
                return carry

            lax.fori_loop(0, VPT, _scan, (prefix,))

            pltpu.sync_copy(lhist_v.at[pl.ds(0, 128)],
                            h_sps[p].at[ident_lo], add=True)
            pltpu.sync_copy(lhist_v.at[pl.ds(128, 128)],
                            h_sps[p].at[ident_hi], add=True)
            plsc.subcore_barrier()
            pltpu.sync_copy(h_sps[p], hback_v)

            def _solve(i, carry):
                accum, found, tb, g_above = carry
                j = 15 - i
                hv = hback_v[pl.ds(j * LANES, LANES)]
                rev = lax.rev(hv, (0,))
                c = plsc.cumsum(rev)
                cond = (accum + c) >= k_rem
                npos = jnp.sum(cond.astype(jnp.int32))
                l = jnp.max(plsc.all_reduce_ffs(cond))
                c_l = jnp.sum(jnp.where(iota == l, c, 0))
                h_l = jnp.sum(jnp.where(iota == (15 - l), hv, 0))
                hit = jnp.logical_and(found == 0, npos > 0)
                tb = jnp.where(hit, j * LANES + 15 - l, tb)
                g_above = jnp.where(hit, accum + c_l - h_l, g_above)
                found = jnp.where(hit, jnp.int32(1), found)
                accum = accum + jnp.sum(hv)
                return accum, found, tb, g_above

            _, _, tb, g_above = lax.fori_loop(
                0, 16, _solve,
                (jnp.int32(0), jnp.int32(0), jnp.int32(0), jnp.int32(0)))
            if p == 0:
                prefix = tb
            else:
                prefix = lax.shift_left(prefix, 8) | tb
            k_rem = k_rem - g_above

        t_b = prefix
        t_s = t_b ^ MIN_I32

        def _cnt(i, carry):
            cg, ce = carry
            kb = keys_v[pl.ds(i * LANES, LANES)]
            gt = (kb ^ MIN_I32) > t_s
            eq = kb == t_b
            return (cg + jnp.sum(gt.astype(jnp.int32)),
                    ce + jnp.sum(eq.astype(jnp.int32)))

        cg, ce = lax.fori_loop(0, VPT, _cnt, (jnp.int32(0), jnp.int32(0)))
        row_v[...] = jnp.where(iota == 0, cg, jnp.where(iota == 1, ce, 0))
        cidx_v[...] = jnp.where(iota == 0, wid,
                                jnp.where(iota == 1, NTILE + wid,
                                          2 * NTILE + iota))
        pltpu.sync_copy(row_v, cnts_sp.at[cidx_v], add=True)
        plsc.subcore_barrier()
        pltpu.sync_copy(cnts_sp, cnts_v)

        cntg = cnts_v[pl.ds(0, LANES)]
        cnte = cnts_v[pl.ds(LANES, LANES)]
        need_eq = jnp.int32(K) - jnp.sum(cntg)
        gt_before = jnp.sum(jnp.where(iota == wid, plsc.cumsum(cntg), 0)) - cg
        eq_before = jnp.sum(jnp.where(iota == wid, plsc.cumsum(cnte), 0)) - ce
        my_off = gt_before + jnp.minimum(eq_before, need_eq)
        my_eq_budget = jnp.clip(need_eq - eq_before, 0, ce)
        trash = jnp.int32(K) + wid

        def _tgt(i, carry, buf_i, buf_v, voff):
            off, eq_taken = carry
            kb = keys_v[pl.ds((i + voff) * LANES, LANES)]
            gt = (kb ^ MIN_I32) > t_s
            eq = kb == t_b
            eqc = plsc.cumsum(eq.astype(jnp.int32))
            take_eq = jnp.logical_and(eq, eqc + eq_taken <= my_eq_budget)
            m = jnp.logical_or(gt, take_eq)
            mi = m.astype(jnp.int32)
            rank = plsc.cumsum(mi)
            buf_i[pl.ds(i * LANES, LANES)] = jnp.where(m, off + rank - 1,
                                                       trash)
            buf_v[pl.ds(i * LANES, LANES)] = iota + (base + (i + voff) * LANES)
            return (off + jnp.sum(mi),
                    eq_taken + jnp.sum(take_eq.astype(jnp.int32)))

        carry = lax.fori_loop(
            0, 8, functools.partial(_tgt, buf_i=idxa_v, buf_v=vala_v, voff=0),
            (my_off, jnp.int32(0)))
        lax.fori_loop(
            0, 8, functools.partial(_tgt, buf_i=idxb_v, buf_v=valb_v, voff=8),
            carry)

        pltpu.sync_copy(vala_v, out_sp.at[idxa_v])
        pltpu.sync_copy(valb_v, out_sp.at[idxb_v])
        plsc.subcore_barrier()

        @pl.when(wid == 0)
        def _flush():
            pltpu.sync_copy(out_sp, out_hbm)


@functools.lru_cache(maxsize=None)
def _get_topk_sc():
    return pl.kernel(
        _topk_body,
        out_type=jax.ShapeDtypeStruct((OUT_PAD,), jnp.int32),
        mesh=plsc.VectorSubcoreMesh(core_axis_name="c", subcore_axis_name="s"),
        compiler_params=pltpu.CompilerParams(needs_layout_passes=False),
        scratch_types=[
            pltpu.VMEM((EPT,), jnp.float32),
            pltpu.VMEM((EPT,), jnp.int32),
            pltpu.VMEM((256,), jnp.int32),
            pltpu.VMEM((256,), jnp.int32),
            pltpu.VMEM((256,), jnp.int32),
            pltpu.VMEM((LANES,), jnp.int32),
            pltpu.VMEM((LANES,), jnp.int32),
            pltpu.VMEM((3 * NTILE,), jnp.int32),
            pltpu.VMEM((128,), jnp.int32),
            pltpu.VMEM((128,), jnp.int32),
            pltpu.VMEM((128,), jnp.int32),
            pltpu.VMEM((128,), jnp.int32),
            pltpu.VMEM((128,), jnp.int32),
            pltpu.VMEM((128,), jnp.int32),
            pltpu.VMEM_SHARED((256,), jnp.int32),
            pltpu.VMEM_SHARED((256,), jnp.int32),
            pltpu.VMEM_SHARED((256,), jnp.int32),
            pltpu.VMEM_SHARED((256,), jnp.int32),
            pltpu.VMEM_SHARED((3 * NTILE,), jnp.int32),
            pltpu.VMEM_SHARED((OUT_PAD,), jnp.int32),
        ],
    )


def kernel(H, w):
    score = _scores_tc(H, jnp.asarray(w, jnp.float32)).reshape(N)
    return _get_topk_sc()(score)[:K]
